# baseline (speedup 1.0000x reference)
import functools

import jax
import jax.numpy as jnp
from jax import lax
from jax.experimental import pallas as pl
from jax.experimental.pallas import tpu as pltpu
from jax.experimental.pallas import tpu_sc as plsc

_EPS = 1e-7
_B = 16
_N = 22743
_C = 85
_ROWS = _B * _N
_PACKED_ROWS = _ROWS // 16
_H0 = (_N + 1) // 2
_H1 = _N - _H0


def _sc_compact_body(x_ref, t_ref, ox_ref, ot_ref, sem_x, sem_t):
    c = lax.axis_index("c")
    s = lax.axis_index("s")

    @pl.when(c == 0)
    def _half0():
        cp_x = pltpu.make_async_copy(
            x_ref.at[s, pl.ds(0, _H0), pl.ds(0, 8)],
            ox_ref.at[pl.ds(s * _N, _H0), :], sem_x)
        cp_t = pltpu.make_async_copy(
            t_ref.at[s, pl.ds(0, _H0), pl.ds(0, 8)],
            ot_ref.at[pl.ds(s * _N, _H0), :], sem_t)
        cp_x.start()
        cp_t.start()
        cp_x.wait()
        cp_t.wait()

    @pl.when(c == 1)
    def _half1():
        cp_x = pltpu.make_async_copy(
            x_ref.at[s, pl.ds(_H0, _H1), pl.ds(0, 8)],
            ox_ref.at[pl.ds(s * _N + _H0, _H1), :], sem_x)
        cp_t = pltpu.make_async_copy(
            t_ref.at[s, pl.ds(_H0, _H1), pl.ds(0, 8)],
            ot_ref.at[pl.ds(s * _N + _H0, _H1), :], sem_t)
        cp_x.start()
        cp_t.start()
        cp_x.wait()
        cp_t.wait()


@functools.partial(
    pl.kernel,
    mesh=plsc.VectorSubcoreMesh(core_axis_name="c", subcore_axis_name="s"),
    out_type=[
        jax.ShapeDtypeStruct((_ROWS, 8), jnp.float32),
        jax.ShapeDtypeStruct((_ROWS, 8), jnp.float32),
    ],
    scratch_types=[pltpu.SemaphoreType.DMA, pltpu.SemaphoreType.DMA],
    compiler_params=pltpu.CompilerParams(use_tc_tiling_on_sc=False),
)
def _sc_compact(x_hbm, t_hbm, ox_hbm, ot_hbm, sem_x, sem_t):
    _sc_compact_body(x_hbm, t_hbm, ox_hbm, ot_hbm, sem_x, sem_t)


_TC_ROWS = 1024


def _tc_loss_body(x_ref, t_ref, out_ref, acc_ref):
    i = pl.program_id(0)

    @pl.when(i == 0)
    def _init():
        acc_ref[0] = 0.0
        acc_ref[1] = 0.0

    xp = x_ref[...]
    tp = t_ref[...]
    lane8 = jax.lax.broadcasted_iota(jnp.int32, (_TC_ROWS, 128), 1) % 8
    rowq = jax.lax.broadcasted_iota(jnp.int32, (_TC_ROWS, 128), 0)
    valid = rowq < (_PACKED_ROWS - i * _TC_ROWS)

    p = jnp.clip(xp, _EPS, 1.0 - _EPS)
    bce = -(tp * jnp.log(p) + (1.0 - tp) * jnp.log(1.0 - p))

    b = jnp.where(valid & (lane8 == 4), (tp > 0.0).astype(jnp.float32), 0.0)
    mb = (jnp.roll(b, -1, axis=1) + jnp.roll(b, -2, axis=1)
          + jnp.roll(b, -3, axis=1) + jnp.roll(b, -4, axis=1))
    val = jnp.where(valid & (lane8 < 4), bce, 0.0) * mb

    acc_ref[0] += jnp.sum(val)
    acc_ref[1] += jnp.sum(b)

    @pl.when(i == pl.num_programs(0) - 1)
    def _fin():
        out_ref[0, 0] = acc_ref[0] / jnp.maximum(acc_ref[1] * 2.0, 1.0)


def _tc_loss(px, pt):
    return pl.pallas_call(
        _tc_loss_body,
        grid=((_PACKED_ROWS + _TC_ROWS - 1) // _TC_ROWS,),
        in_specs=[
            pl.BlockSpec((_TC_ROWS, 128), lambda i: (i, 0)),
            pl.BlockSpec((_TC_ROWS, 128), lambda i: (i, 0)),
        ],
        out_specs=pl.BlockSpec(memory_space=pltpu.SMEM),
        out_shape=jax.ShapeDtypeStruct((1, 1), jnp.float32),
        scratch_shapes=[pltpu.SMEM((2,), jnp.float32)],
        compiler_params=pltpu.CompilerParams(
            dimension_semantics=("arbitrary",),
        ),
    )(px, pt)


def kernel(x, target):
    px, pt = _sc_compact(x, target)
    px2 = px.reshape(_PACKED_ROWS, 128)
    pt2 = pt.reshape(_PACKED_ROWS, 128)
    return _tc_loss(px2, pt2)[0, 0]


# speedup vs baseline: 10.7545x; 10.7545x over previous
import functools

import jax
import jax.numpy as jnp
from jax import lax
from jax.experimental import pallas as pl
from jax.experimental.pallas import tpu as pltpu
from jax.experimental.pallas import tpu_sc as plsc

_EPS = 1e-7
_B = 16
_N = 22743
_C = 85
_HALF0 = 11392
_CHUNK = 2048
_ROWS_PER_HALF = _HALF0 // 16
_OUT_ROWS = 2 * _ROWS_PER_HALF


_SUB = 256


def _emit_half(s, h, wid, x_ref, t_ref, o_ref, rx, rt, bx, bt, sgx, sgt, ssx, sst):
    n0 = _HALF0 * h
    n_anchors = _HALF0 if h == 0 else _N - _HALF0
    for k in range(6):
        c0 = k * _CHUNK
        clen = min(_CHUNK, n_anchors - c0)
        for m in range(_CHUNK // _SUB):
            mlen = min(_SUB, clen - m * _SUB)
            if mlen <= 0:
                continue
            g_x = pltpu.make_async_copy(
                x_ref.at[s, pl.ds(n0 + c0 + m * _SUB, mlen), :],
                rx.at[pl.ds(0, mlen), :], sgx)
            g_t = pltpu.make_async_copy(
                t_ref.at[s, pl.ds(n0 + c0 + m * _SUB, mlen), :],
                rt.at[pl.ds(0, mlen), :], sgt)
            g_x.start()
            g_t.start()
            g_x.wait()
            g_t.wait()
            for jj in range(_SUB // 128):
                jlen = min(128, mlen - jj * 128)
                if jlen <= 0:
                    continue
                j = (_SUB // 128) * m + jj
                pltpu.sync_copy(
                    rx.at[pl.ds(128 * jj, jlen), pl.ds(0, 8)],
                    bx.at[wid, pl.ds(0, jlen), pl.ds(8 * j, 8)])
                pltpu.sync_copy(
                    rt.at[pl.ds(128 * jj, jlen), pl.ds(0, 8)],
                    bt.at[wid, pl.ds(0, jlen), pl.ds(8 * j, 8)])
        out_row = h * _ROWS_PER_HALF + k * 128
        nrows = (clen + 15) // 16
        nrows = (nrows + 7) // 8 * 8
        sc_x = pltpu.make_async_copy(
            bx.at[wid, pl.ds(0, nrows), :],
            o_ref.at[0, s, pl.ds(out_row, nrows), :], ssx)
        sc_t = pltpu.make_async_copy(
            bt.at[wid, pl.ds(0, nrows), :],
            o_ref.at[1, s, pl.ds(out_row, nrows), :], sst)
        sc_x.start()
        sc_t.start()
        sc_x.wait()
        sc_t.wait()


@functools.partial(
    pl.kernel,
    mesh=plsc.VectorSubcoreMesh(core_axis_name="c", subcore_axis_name="s"),
    out_type=jax.ShapeDtypeStruct((2, _B, _OUT_ROWS, 128), jnp.float32),
    scratch_types=[
        pltpu.VMEM((_SUB, _C), jnp.float32),
        pltpu.VMEM((_SUB, _C), jnp.float32),
        pltpu.VMEM_SHARED((16, 128, 128), jnp.float32),
        pltpu.VMEM_SHARED((16, 128, 128), jnp.float32),
        pltpu.SemaphoreType.DMA,
        pltpu.SemaphoreType.DMA,
        pltpu.SemaphoreType.DMA,
        pltpu.SemaphoreType.DMA,
    ],
)
def _sc_compact(x_hbm, t_hbm, o_hbm, rx, rt, bx, bt, sgx, sgt, ssx, sst):
    c = lax.axis_index("c")
    s = lax.axis_index("s")

    @pl.when(c == 0)
    def _h0():
        _emit_half(s, 0, s, x_hbm, t_hbm, o_hbm, rx, rt, bx, bt,
                   sgx, sgt, ssx, sst)

    @pl.when(c == 1)
    def _h1():
        _emit_half(s, 1, s, x_hbm, t_hbm, o_hbm, rx, rt, bx, bt,
                   sgx, sgt, ssx, sst)


def _tc_loss_body(xt_ref, out_ref, acc_ref, mv_ref, mb_ref):
    i = pl.program_id(0)

    @pl.when(i == 0)
    def _init():
        acc_ref[0] = 0.0
        acc_ref[1] = 0.0
        row = jax.lax.broadcasted_iota(jnp.int32, (_OUT_ROWS, 128), 0)
        lane = jax.lax.broadcasted_iota(jnp.int32, (_OUT_ROWS, 128), 1)
        h = (row >= _ROWS_PER_HALF).astype(jnp.int32)
        rr = row - h * _ROWS_PER_HALF
        a = (_HALF0 * h + (rr // 128) * _CHUNK + (lane // 8) * 128
             + (rr % 128))
        cap = _HALF0 + (_N - _HALF0) * h
        valid = a < cap
        lane8 = lane % 8
        mv_ref[...] = valid.astype(jnp.float32)
        mb_ref[...] = (valid & (lane8 == 4)).astype(jnp.float32)

    xp = xt_ref[0, 0]
    tp = xt_ref[1, 0]
    ok = mv_ref[...] > 0.0

    pq = jnp.clip(jnp.where(ok, xp, 0.5), _EPS, 1.0 - _EPS)
    tq = jnp.where(ok, tp, 0.5)
    bce = -(tq * jnp.log(pq) + (1.0 - tq) * jnp.log(1.0 - pq))

    b = mb_ref[...] * (tq > 0.0).astype(jnp.float32)
    mb = (jnp.roll(b, -1, axis=1) + jnp.roll(b, -2, axis=1)
          + jnp.roll(b, -3, axis=1) + jnp.roll(b, -4, axis=1))

    acc_ref[0] += jnp.sum(bce * mb)
    acc_ref[1] += jnp.sum(b)

    @pl.when(i == pl.num_programs(0) - 1)
    def _fin():
        out_ref[0, 0] = acc_ref[0] / jnp.maximum(acc_ref[1] * 2.0, 1.0)


def _tc_loss(packed):
    return pl.pallas_call(
        _tc_loss_body,
        grid=(_B,),
        in_specs=[
            pl.BlockSpec((2, 1, _OUT_ROWS, 128), lambda i: (0, i, 0, 0)),
        ],
        out_specs=pl.BlockSpec(memory_space=pltpu.SMEM),
        out_shape=jax.ShapeDtypeStruct((1, 1), jnp.float32),
        scratch_shapes=[
            pltpu.SMEM((2,), jnp.float32),
            pltpu.VMEM((_OUT_ROWS, 128), jnp.float32),
            pltpu.VMEM((_OUT_ROWS, 128), jnp.float32),
        ],
        compiler_params=pltpu.CompilerParams(
            dimension_semantics=("arbitrary",),
        ),
    )(packed)


def kernel(x, target):
    packed = _sc_compact(x, target)
    return _tc_loss(packed)[0, 0]


# speedup vs baseline: 13.1455x; 1.2223x over previous
import functools

import jax
import jax.numpy as jnp
from jax import lax
from jax.experimental import pallas as pl
from jax.experimental.pallas import tpu as pltpu
from jax.experimental.pallas import tpu_sc as plsc

_EPS = 1e-7
_B = 16
_N = 22743
_C = 85
_HALF0 = 11392
_CHUNK = 2048
_ROWS_PER_HALF = _HALF0 // 16
_OUT_ROWS = 2 * _ROWS_PER_HALF


def _emit_half(s, h, wid, x_ref, t_ref, o_ref, rbufs, bx, bt,
               sgx, sgt, ssx, sst):
    n0 = _HALF0 * h
    n_anchors = _HALF0 if h == 0 else _N - _HALF0
    schedule = []
    for k in range(6):
        clen = min(_CHUNK, n_anchors - k * _CHUNK)
        nm = (clen + 127) // 128
        for m in range(nm):
            schedule.append((k, m, min(128, clen - 128 * m), m == nm - 1,
                             (clen + 15) // 16))

    def issue(i):
        k, m, slen, _, _ = schedule[i]
        rx, rt = rbufs[i % 2]
        g_x = pltpu.make_async_copy(
            x_ref.at[s, pl.ds(n0 + _CHUNK * k + 128 * m, slen), :],
            rx.at[pl.ds(0, slen), :], sgx)
        g_t = pltpu.make_async_copy(
            t_ref.at[s, pl.ds(n0 + _CHUNK * k + 128 * m, slen), :],
            rt.at[pl.ds(0, slen), :], sgt)
        g_x.start()
        g_t.start()
        return g_x, g_t

    pending = issue(0)
    out_pending = []
    for i, (k, m, slen, last, crows) in enumerate(schedule):
        g_x, g_t = pending
        if i + 1 < len(schedule):
            pending = issue(i + 1)
        g_x.wait()
        g_t.wait()
        if m == 0:
            for cp in out_pending:
                cp.wait()
            out_pending = []
        rx, rt = rbufs[i % 2]
        pltpu.sync_copy(rx.at[pl.ds(0, slen), pl.ds(0, 8)],
                        bx.at[wid, pl.ds(0, slen), pl.ds(8 * m, 8)])
        pltpu.sync_copy(rt.at[pl.ds(0, slen), pl.ds(0, 8)],
                        bt.at[wid, pl.ds(0, slen), pl.ds(8 * m, 8)])
        if last:
            out_row = h * _ROWS_PER_HALF + k * 128
            nrows = (crows + 7) // 8 * 8
            sc_x = pltpu.make_async_copy(
                bx.at[wid, pl.ds(0, nrows), :],
                o_ref.at[0, s, pl.ds(out_row, nrows), :], ssx)
            sc_t = pltpu.make_async_copy(
                bt.at[wid, pl.ds(0, nrows), :],
                o_ref.at[1, s, pl.ds(out_row, nrows), :], sst)
            sc_x.start()
            sc_t.start()
            out_pending = [sc_x, sc_t]
    for cp in out_pending:
        cp.wait()


@functools.partial(
    pl.kernel,
    mesh=plsc.VectorSubcoreMesh(core_axis_name="c", subcore_axis_name="s"),
    out_type=jax.ShapeDtypeStruct((2, _B, _OUT_ROWS, 128), jnp.float32),
    scratch_types=[
        pltpu.VMEM((128, _C), jnp.float32),
        pltpu.VMEM((128, _C), jnp.float32),
        pltpu.VMEM((128, _C), jnp.float32),
        pltpu.VMEM((128, _C), jnp.float32),
        pltpu.VMEM_SHARED((16, 128, 128), jnp.float32),
        pltpu.VMEM_SHARED((16, 128, 128), jnp.float32),
        pltpu.SemaphoreType.DMA,
        pltpu.SemaphoreType.DMA,
        pltpu.SemaphoreType.DMA,
        pltpu.SemaphoreType.DMA,
    ],
)
def _sc_compact(x_hbm, t_hbm, o_hbm, rx0, rt0, rx1, rt1, bx, bt,
                sgx, sgt, ssx, sst):
    c = lax.axis_index("c")
    s = lax.axis_index("s")
    rbufs = [(rx0, rt0), (rx1, rt1)]

    @pl.when(c == 0)
    def _h0():
        _emit_half(s, 0, s, x_hbm, t_hbm, o_hbm, rbufs, bx, bt,
                   sgx, sgt, ssx, sst)

    @pl.when(c == 1)
    def _h1():
        _emit_half(s, 1, s, x_hbm, t_hbm, o_hbm, rbufs, bx, bt,
                   sgx, sgt, ssx, sst)


def _tc_loss_body(xt_ref, out_ref, acc_ref, mv_ref, mb_ref):
    i = pl.program_id(0)

    @pl.when(i == 0)
    def _init():
        acc_ref[0] = 0.0
        acc_ref[1] = 0.0
        row = jax.lax.broadcasted_iota(jnp.int32, (_OUT_ROWS, 128), 0)
        lane = jax.lax.broadcasted_iota(jnp.int32, (_OUT_ROWS, 128), 1)
        h = (row >= _ROWS_PER_HALF).astype(jnp.int32)
        rr = row - h * _ROWS_PER_HALF
        a = (_HALF0 * h + (rr // 128) * _CHUNK + (lane // 8) * 128
             + (rr % 128))
        cap = _HALF0 + (_N - _HALF0) * h
        valid = a < cap
        lane8 = lane % 8
        mv_ref[...] = valid.astype(jnp.float32)
        mb_ref[...] = (valid & (lane8 == 4)).astype(jnp.float32)

    xp = xt_ref[0, 0]
    tp = xt_ref[1, 0]
    ok = mv_ref[...] > 0.0

    pq = jnp.clip(jnp.where(ok, xp, 0.5), _EPS, 1.0 - _EPS)
    tq = jnp.where(ok, tp, 0.5)
    bce = -(tq * jnp.log(pq) + (1.0 - tq) * jnp.log(1.0 - pq))

    b = mb_ref[...] * (tq > 0.0).astype(jnp.float32)
    mb = (jnp.roll(b, -1, axis=1) + jnp.roll(b, -2, axis=1)
          + jnp.roll(b, -3, axis=1) + jnp.roll(b, -4, axis=1))

    acc_ref[0] += jnp.sum(bce * mb)
    acc_ref[1] += jnp.sum(b)

    @pl.when(i == pl.num_programs(0) - 1)
    def _fin():
        out_ref[0, 0] = acc_ref[0] / jnp.maximum(acc_ref[1] * 2.0, 1.0)


def _tc_loss(packed):
    return pl.pallas_call(
        _tc_loss_body,
        grid=(_B,),
        in_specs=[
            pl.BlockSpec((2, 1, _OUT_ROWS, 128), lambda i: (0, i, 0, 0)),
        ],
        out_specs=pl.BlockSpec(memory_space=pltpu.SMEM),
        out_shape=jax.ShapeDtypeStruct((1, 1), jnp.float32),
        scratch_shapes=[
            pltpu.SMEM((2,), jnp.float32),
            pltpu.VMEM((_OUT_ROWS, 128), jnp.float32),
            pltpu.VMEM((_OUT_ROWS, 128), jnp.float32),
        ],
        compiler_params=pltpu.CompilerParams(
            dimension_semantics=("arbitrary",),
        ),
    )(packed)


def kernel(x, target):
    packed = _sc_compact(x, target)
    return _tc_loss(packed)[0, 0]


# speedup vs baseline: 13.9135x; 1.0584x over previous
import functools

import jax
import jax.numpy as jnp
from jax import lax
from jax.experimental import pallas as pl
from jax.experimental.pallas import tpu as pltpu
from jax.experimental.pallas import tpu_sc as plsc

_EPS = 1e-7
_B = 16
_N = 22743
_C = 85
_CHUNK = 2048
_OUT_ROWS = 1424
_QSTART = (0, 5632, 11392, 17024)
_QEND = (5632, 11392, 17024, _N)


def _emit_range(s, slot, astart, aend, x_ref, t_ref, o_ref, rbufs, bx, bt,
                sgx, sgt, ssx, sst):
    alen = aend - astart
    schedule = []
    nk = (alen + _CHUNK - 1) // _CHUNK
    for k in range(nk):
        clen = min(_CHUNK, alen - k * _CHUNK)
        nm = (clen + 127) // 128
        for m in range(nm):
            schedule.append((k, m, min(128, clen - 128 * m), m == nm - 1,
                             (clen + 15) // 16))

    def issue(i):
        k, m, slen, _, _ = schedule[i]
        rx, rt = rbufs[i % 2]
        g_x = pltpu.make_async_copy(
            x_ref.at[s, pl.ds(astart + _CHUNK * k + 128 * m, slen), :],
            rx.at[pl.ds(0, slen), :], sgx)
        g_t = pltpu.make_async_copy(
            t_ref.at[s, pl.ds(astart + _CHUNK * k + 128 * m, slen), :],
            rt.at[pl.ds(0, slen), :], sgt)
        g_x.start()
        g_t.start()
        return g_x, g_t

    wid = s - 8
    pending = issue(0)
    out_pending = []
    for i, (k, m, slen, last, crows) in enumerate(schedule):
        g_x, g_t = pending
        if i + 1 < len(schedule):
            pending = issue(i + 1)
        g_x.wait()
        g_t.wait()
        if m == 0:
            for cp in out_pending:
                cp.wait()
            out_pending = []
        rx, rt = rbufs[i % 2]
        pltpu.sync_copy(rx.at[pl.ds(0, slen), pl.ds(0, 8)],
                        bx.at[slot, pl.ds(0, slen), pl.ds(8 * m, 8)])
        pltpu.sync_copy(rt.at[pl.ds(0, slen), pl.ds(0, 8)],
                        bt.at[slot, pl.ds(0, slen), pl.ds(8 * m, 8)])
        if last:
            out_row = astart // 16 + k * 128
            nrows = (crows + 7) // 8 * 8
            sc_x = pltpu.make_async_copy(
                bx.at[slot, pl.ds(0, nrows), :],
                o_ref.at[0, wid, pl.ds(out_row, nrows), :], ssx)
            sc_t = pltpu.make_async_copy(
                bt.at[slot, pl.ds(0, nrows), :],
                o_ref.at[1, wid, pl.ds(out_row, nrows), :], sst)
            sc_x.start()
            sc_t.start()
            out_pending = [sc_x, sc_t]
    for cp in out_pending:
        cp.wait()


@functools.partial(
    pl.kernel,
    mesh=plsc.VectorSubcoreMesh(core_axis_name="c", subcore_axis_name="s"),
    out_type=jax.ShapeDtypeStruct((2, 8, _OUT_ROWS, 128), jnp.float32),
    scratch_types=[
        pltpu.VMEM((128, _C), jnp.float32),
        pltpu.VMEM((128, _C), jnp.float32),
        pltpu.VMEM((128, _C), jnp.float32),
        pltpu.VMEM((128, _C), jnp.float32),
        pltpu.VMEM_SHARED((16, 128, 128), jnp.float32),
        pltpu.VMEM_SHARED((16, 128, 128), jnp.float32),
        pltpu.SemaphoreType.DMA,
        pltpu.SemaphoreType.DMA,
        pltpu.SemaphoreType.DMA,
        pltpu.SemaphoreType.DMA,
    ],
)
def _sc_compact(x_hbm, t_hbm, o_hbm, rx0, rt0, rx1, rt1, bx, bt,
                sgx, sgt, ssx, sst):
    c = lax.axis_index("c")
    s = lax.axis_index("s")
    rbufs = [(rx0, rt0), (rx1, rt1)]

    for qbase, smin in ((0, 0), (2, 8)):
        for ci in range(2):
            q = qbase + ci

            @pl.when((c == ci) & (s >= smin) & (s < smin + 8))
            def _go(q=q, smin=smin):
                batch = s + (8 - smin)
                _emit_range(batch, s, _QSTART[q], _QEND[q], x_hbm, t_hbm,
                            o_hbm, rbufs, bx, bt, sgx, sgt, ssx, sst)


def _bce_partials(xp, tp, ok, mb_sel):
    pq = jnp.clip(jnp.where(ok, xp, 0.5), _EPS, 1.0 - _EPS)
    tq = jnp.where(ok, tp, 0.5)
    bce = -(tq * jnp.log(pq) + (1.0 - tq) * jnp.log(1.0 - pq))
    b = mb_sel * (tq > 0.0).astype(jnp.float32)
    mb = (jnp.roll(b, -1, axis=1) + jnp.roll(b, -2, axis=1)
          + jnp.roll(b, -3, axis=1) + jnp.roll(b, -4, axis=1))
    return jnp.sum(bce * mb), jnp.sum(b)


def _tc_direct_body(x_ref, t_ref, out_ref, acc_ref):
    bi = pl.program_id(0)
    kb = pl.program_id(1)

    @pl.when((bi == 0) & (kb == 0))
    def _init():
        acc_ref[0] = 0.0
        acc_ref[1] = 0.0

    xb = x_ref[0]
    tb = t_ref[0]
    xp = jnp.concatenate(
        [xb[128 * j:128 * (j + 1), 0:8] for j in range(16)], axis=1)
    tp = jnp.concatenate(
        [tb[128 * j:128 * (j + 1), 0:8] for j in range(16)], axis=1)

    lane = jax.lax.broadcasted_iota(jnp.int32, (128, 128), 1)
    q = jax.lax.broadcasted_iota(jnp.int32, (128, 128), 0)
    a = kb * _CHUNK + (lane // 8) * 128 + q
    valid = a < _N
    ok = valid
    mb_sel = (valid & (lane % 8 == 4)).astype(jnp.float32)

    ds, db = _bce_partials(xp, tp, ok, mb_sel)
    acc_ref[0] += ds
    acc_ref[1] += db

    @pl.when((bi == pl.num_programs(0) - 1) & (kb == pl.num_programs(1) - 1))
    def _fin():
        out_ref[0] = acc_ref[0]
        out_ref[1] = acc_ref[1]


def _tc_direct(x, target):
    nkb = (_N + _CHUNK - 1) // _CHUNK
    return pl.pallas_call(
        _tc_direct_body,
        grid=(8, nkb),
        in_specs=[
            pl.BlockSpec((1, _CHUNK, _C), lambda b, k: (b, k, 0)),
            pl.BlockSpec((1, _CHUNK, _C), lambda b, k: (b, k, 0)),
        ],
        out_specs=pl.BlockSpec(memory_space=pltpu.SMEM),
        out_shape=jax.ShapeDtypeStruct((2,), jnp.float32),
        scratch_shapes=[pltpu.SMEM((2,), jnp.float32)],
        compiler_params=pltpu.CompilerParams(
            dimension_semantics=("arbitrary", "arbitrary"),
        ),
    )(x, target)


def _tc_packed_body(xt_ref, part_ref, out_ref, acc_ref, mv_ref, mb_ref):
    i = pl.program_id(0)

    @pl.when(i == 0)
    def _init():
        acc_ref[0] = part_ref[0]
        acc_ref[1] = part_ref[1]
        row = jax.lax.broadcasted_iota(jnp.int32, (_OUT_ROWS, 128), 0)
        lane = jax.lax.broadcasted_iota(jnp.int32, (_OUT_ROWS, 128), 1)
        astart = jnp.zeros_like(row)
        base = jnp.zeros_like(row)
        aend = jnp.full_like(row, _QEND[0])
        for qi in range(1, 4):
            sel = row >= _QSTART[qi] // 16
            astart = jnp.where(sel, _QSTART[qi], astart)
            base = jnp.where(sel, _QSTART[qi] // 16, base)
            aend = jnp.where(sel, _QEND[qi], aend)
        rr = row - base
        a = astart + (rr // 128) * _CHUNK + (lane // 8) * 128 + (rr % 128)
        valid = a < aend
        mv_ref[...] = valid.astype(jnp.float32)
        mb_ref[...] = (valid & (lane % 8 == 4)).astype(jnp.float32)

    ok = mv_ref[...] > 0.0
    ds, db = _bce_partials(xt_ref[0, 0], xt_ref[1, 0], ok, mb_ref[...])
    acc_ref[0] += ds
    acc_ref[1] += db

    @pl.when(i == pl.num_programs(0) - 1)
    def _fin():
        out_ref[0, 0] = acc_ref[0] / jnp.maximum(acc_ref[1] * 2.0, 1.0)


def _tc_packed(packed, partials):
    return pl.pallas_call(
        _tc_packed_body,
        grid=(8,),
        in_specs=[
            pl.BlockSpec((2, 1, _OUT_ROWS, 128), lambda i: (0, i, 0, 0)),
            pl.BlockSpec(memory_space=pltpu.SMEM),
        ],
        out_specs=pl.BlockSpec(memory_space=pltpu.SMEM),
        out_shape=jax.ShapeDtypeStruct((1, 1), jnp.float32),
        scratch_shapes=[
            pltpu.SMEM((2,), jnp.float32),
            pltpu.VMEM((_OUT_ROWS, 128), jnp.float32),
            pltpu.VMEM((_OUT_ROWS, 128), jnp.float32),
        ],
        compiler_params=pltpu.CompilerParams(
            dimension_semantics=("arbitrary",),
        ),
    )(packed, partials)


def kernel(x, target):
    packed = _sc_compact(x, target)
    partials = _tc_direct(x, target)
    return _tc_packed(packed, partials)[0, 0]
